# baseline (speedup 1.0000x reference)
import functools

import jax
import jax.numpy as jnp
from jax import lax
from jax.experimental import pallas as pl
from jax.experimental.pallas import tpu as pltpu
from jax.experimental.pallas import tpu_sc as plsc

N = 10000
E = 320000
D = 128
G = 64
H = 64
OUT = 10

NC = 2
NS = 16
L = 16
NW = NC * NS
EW = E // NW
K = 80
NCH = EW // K
KA = 100
NCHA = EW // KA
CPB = 4
NBLK = NCHA // CPB

ZRT = 624
ZB = 8

TB = 1000
NB = N // TB

_mesh = plsc.VectorSubcoreMesh(core_axis_name="c", subcore_axis_name="s")


@functools.partial(
    pl.kernel,
    out_type=jax.ShapeDtypeStruct((NW, N), jnp.float32),
    mesh=_mesh,
    scratch_types=[
        pltpu.VMEM((NCH, K), jnp.int32),
        pltpu.VMEM((N,), jnp.float32),
    ],
    compiler_params=pltpu.CompilerParams(needs_layout_passes=False),
)
def _sc_degree(dst_hbm, out_hbm, idx_d, degv):
    c = lax.axis_index("c")
    s = lax.axis_index("s")
    wid = s * NC + c
    z16 = jnp.zeros((L,), jnp.float32)
    ones16 = jnp.ones((L,), jnp.float32)

    def zero_row(i, carry):
        degv[pl.ds(i * L, L)] = z16
        return carry

    lax.fori_loop(0, N // L, zero_row, 0)
    pltpu.sync_copy(dst_hbm.at[wid], idx_d)

    def count_chunk(j, carry):
        for t in range(K // L):
            idx16 = idx_d[j, pl.ds(t * L, L)]
            plsc.addupdate_scatter(degv, [idx16], ones16)
        return carry

    lax.fori_loop(0, NCH, count_chunk, 0)
    pltpu.sync_copy(degv, out_hbm.at[wid])


@functools.partial(
    pl.kernel,
    out_type=jax.ShapeDtypeStruct((NC, N, D), jnp.float32),
    mesh=_mesh,
    scratch_types=[
        pltpu.VMEM((3, 2, CPB, KA), jnp.int32),
        pltpu.VMEM((3, KA, D), jnp.float32),
        pltpu.VMEM((ZB, D), jnp.float32),
        pltpu.VMEM_SHARED((N, D), jnp.float32),
        pltpu.SemaphoreType.DMA,
        pltpu.SemaphoreType.DMA,
        pltpu.SemaphoreType.DMA,
    ],
)
def _sc_aggregate(g_hbm, src_hbm, dst_hbm, out_hbm, idx,
                  rows, zbuf, aggS, gsem, ssem, isem):
    c = lax.axis_index("c")
    s = lax.axis_index("s")
    wid = s * NC + c
    z16 = jnp.zeros((L,), jnp.float32)

    def zero_row(i, carry):
        for j in range(D // L):
            zbuf[i, pl.ds(j * L, L)] = z16
        return carry

    lax.fori_loop(0, ZB, zero_row, 0)
    base = s * ZRT

    def zero_acc(t, carry):
        pltpu.sync_copy(zbuf, aggS.at[pl.ds(base + t * ZB, ZB)])
        return carry

    lax.fori_loop(0, ZRT // ZB, zero_acc, 0)

    @pl.when(s == NS - 1)
    def _():
        for t in range((N - NS * ZRT) // ZB):
            pltpu.sync_copy(zbuf, aggS.at[pl.ds(NS * ZRT + t * ZB, ZB)])

    pltpu.sync_copy(src_hbm.at[wid, 0], idx.at[0, 0])
    pltpu.sync_copy(dst_hbm.at[wid, 0], idx.at[0, 1])
    plsc.subcore_barrier()

    def _m3(v):
        return v - 3 * ((v * 43691) >> 17)

    def _gather(j, rb):
        return pltpu.make_async_copy(
            g_hbm.at[idx.at[_m3(j >> 2), 0, j & (CPB - 1)]], rows.at[rb], gsem)

    def _scatter(j, rb):
        return pltpu.make_async_copy(
            rows.at[rb], aggS.at[idx.at[_m3(j >> 2), 1, j & (CPB - 1)]], ssem)

    def _idx_load_s(b):
        return pltpu.make_async_copy(src_hbm.at[wid, b], idx.at[_m3(b), 0],
                                     isem)

    def _idx_load_d(b):
        return pltpu.make_async_copy(dst_hbm.at[wid, b], idx.at[_m3(b), 1],
                                     isem)

    _gather(0, 0).start()
    _gather(1, 1).start()

    def step(j, carry):
        b = j >> 2
        jj = j & (CPB - 1)
        rb = _m3(j)
        rb2 = _m3(j + 2)

        @pl.when((jj == 0) & (b < NBLK - 1))
        def _():
            _idx_load_s(b + 1).start()
            _idx_load_d(b + 1).start()

        _gather(j, rb).wait()
        pltpu.async_copy(rows.at[rb], aggS.at[idx.at[_m3(b), 1, jj]], ssem,
                         add=True)

        @pl.when(j >= 1)
        def _():
            _scatter(j - 1, rb2).wait()

        @pl.when((jj == 2) & (b < NBLK - 1))
        def _():
            _idx_load_s(b + 1).wait()
            _idx_load_d(b + 1).wait()

        @pl.when(j + 2 < NCHA)
        def _():
            _gather(j + 2, rb2).start()

        return carry

    lax.fori_loop(0, NCHA, step, 0)
    _scatter(NCHA - 1, _m3(NCHA - 1)).wait()
    plsc.subcore_barrier()

    pltpu.sync_copy(aggS.at[pl.ds(base, ZRT)],
                    out_hbm.at[c, pl.ds(base, ZRT)])

    @pl.when(s == NS - 1)
    def _():
        pltpu.sync_copy(aggS.at[pl.ds(NS * ZRT, N - NS * ZRT)],
                        out_hbm.at[c, pl.ds(NS * ZRT, N - NS * ZRT)])


def _tc_stage1_body(degT_ref, x_ref, W_ref, h_ref, g_ref, dinv_ref):
    deg = jnp.sum(degT_ref[...], axis=1) + 1.0
    dinv = lax.rsqrt(deg)
    h = jnp.dot(x_ref[...], W_ref[...], preferred_element_type=jnp.float32)
    h_ref[...] = h
    g_ref[...] = h * dinv[:, None]
    dinv_ref[...] = dinv[:, None]


_stage1 = pl.pallas_call(
    _tc_stage1_body,
    grid=(NB,),
    in_specs=[
        pl.BlockSpec((TB, NW), lambda i: (i, 0)),
        pl.BlockSpec((TB, D), lambda i: (i, 0)),
        pl.BlockSpec((D, D), lambda i: (0, 0)),
    ],
    out_specs=[
        pl.BlockSpec((TB, D), lambda i: (i, 0)),
        pl.BlockSpec((TB, D), lambda i: (i, 0)),
        pl.BlockSpec((TB, 1), lambda i: (i, 0)),
    ],
    out_shape=[
        jax.ShapeDtypeStruct((N, D), jnp.float32),
        jax.ShapeDtypeStruct((N, D), jnp.float32),
        jax.ShapeDtypeStruct((N, 1), jnp.float32),
    ],
)


def _tc_mid_body(agg_ref, h_ref, dinv_ref, b_ref, W_ref, ho_ref, go_ref):
    dinv = dinv_ref[...]
    a = agg_ref[0] + agg_ref[1]
    xin = jnp.maximum(
        dinv * a + dinv * dinv * h_ref[...] + b_ref[...][None, :], 0.0)
    h = jnp.dot(xin, W_ref[...], preferred_element_type=jnp.float32)
    ho_ref[...] = h
    go_ref[...] = h * dinv


_mid = pl.pallas_call(
    _tc_mid_body,
    grid=(NB,),
    in_specs=[
        pl.BlockSpec((NC, TB, D), lambda i: (0, i, 0)),
        pl.BlockSpec((TB, D), lambda i: (i, 0)),
        pl.BlockSpec((TB, 1), lambda i: (i, 0)),
        pl.BlockSpec((D,), lambda i: (0,)),
        pl.BlockSpec((D, D), lambda i: (0, 0)),
    ],
    out_specs=[
        pl.BlockSpec((TB, D), lambda i: (i, 0)),
        pl.BlockSpec((TB, D), lambda i: (i, 0)),
    ],
    out_shape=[
        jax.ShapeDtypeStruct((N, D), jnp.float32),
        jax.ShapeDtypeStruct((N, D), jnp.float32),
    ],
)


def _tc_final_body(agg_ref, h_ref, dinv_ref, b_ref, batch_ref, Wm0_ref, bm0_ref,
                   Wm1_ref, bm1_ref, emb_ref, logits_ref, probs_ref,
                   acc_ref, cnt_ref):
    i = pl.program_id(0)
    dinv = dinv_ref[...]
    a = agg_ref[0] + agg_ref[1]
    emb = jnp.maximum(
        dinv * a + dinv * dinv * h_ref[...] + b_ref[...][None, :], 0.0)
    emb_ref[...] = emb

    @pl.when(i == 0)
    def _():
        acc_ref[...] = jnp.zeros_like(acc_ref)
        cnt_ref[...] = jnp.zeros_like(cnt_ref)

    bat = batch_ref[...][:, 0]
    gids = lax.broadcasted_iota(jnp.int32, (G, TB), 0)
    onehot = (bat[None, :] == gids).astype(jnp.float32)
    acc_ref[...] += jnp.dot(onehot, emb, preferred_element_type=jnp.float32)
    cnt_ref[...] += jnp.sum(onehot, axis=1, keepdims=True)

    @pl.when(i == NB - 1)
    def _():
        pooled = acc_ref[...] / jnp.maximum(cnt_ref[...], 1.0)
        z = jnp.dot(pooled, Wm0_ref[...], preferred_element_type=jnp.float32)
        z = z + bm0_ref[...][None, :]
        z = jnp.where(z > 0, z, jnp.exp(z) - 1.0)
        logits = jnp.dot(z, Wm1_ref[...], preferred_element_type=jnp.float32)
        logits = logits + bm1_ref[...][None, :]
        logits_ref[...] = logits
        m = jnp.max(logits, axis=1, keepdims=True)
        e = jnp.exp(logits - m)
        probs_ref[...] = e / jnp.sum(e, axis=1, keepdims=True)


_final = pl.pallas_call(
    _tc_final_body,
    grid=(NB,),
    in_specs=[
        pl.BlockSpec((NC, TB, D), lambda i: (0, i, 0)),
        pl.BlockSpec((TB, D), lambda i: (i, 0)),
        pl.BlockSpec((TB, 1), lambda i: (i, 0)),
        pl.BlockSpec((D,), lambda i: (0,)),
        pl.BlockSpec((TB, 1), lambda i: (i, 0)),
        pl.BlockSpec((D, H), lambda i: (0, 0)),
        pl.BlockSpec((H,), lambda i: (0,)),
        pl.BlockSpec((H, OUT), lambda i: (0, 0)),
        pl.BlockSpec((OUT,), lambda i: (0,)),
    ],
    out_specs=[
        pl.BlockSpec((TB, D), lambda i: (i, 0)),
        pl.BlockSpec((G, OUT), lambda i: (0, 0)),
        pl.BlockSpec((G, OUT), lambda i: (0, 0)),
    ],
    out_shape=[
        jax.ShapeDtypeStruct((N, D), jnp.float32),
        jax.ShapeDtypeStruct((G, OUT), jnp.float32),
        jax.ShapeDtypeStruct((G, OUT), jnp.float32),
    ],
    scratch_shapes=[
        pltpu.VMEM((G, D), jnp.float32),
        pltpu.VMEM((G, D), jnp.float32),
    ],
)


def kernel(x, edge_index, batch, W1, b1, W2, b2, W3, b3, Wm0, bm0, Wm1, bm1):
    src4 = edge_index[0].reshape(NW, NBLK, CPB, KA)
    dst4 = edge_index[1].reshape(NW, NBLK, CPB, KA)
    dstd = edge_index[1].reshape(NW, NCH, K)
    deg = _sc_degree(dstd)
    h1, g1, dinv = _stage1(deg.T, x, W1)
    agg1 = _sc_aggregate(g1, src4, dst4)
    h2, g2 = _mid(agg1, h1, dinv, b1, W2)
    agg2 = _sc_aggregate(g2, src4, dst4)
    h3, g3 = _mid(agg2, h2, dinv, b2, W3)
    agg3 = _sc_aggregate(g3, src4, dst4)
    emb, logits, probs = _final(agg3, h3, dinv, b3, batch.reshape(N, 1),
                                Wm0, bm0, Wm1, bm1)
    return logits, probs, emb


# speedup vs baseline: 1.0152x; 1.0152x over previous
import functools

import jax
import jax.numpy as jnp
from jax import lax
from jax.experimental import pallas as pl
from jax.experimental.pallas import tpu as pltpu
from jax.experimental.pallas import tpu_sc as plsc

N = 10000
E = 320000
D = 128
G = 64
H = 64
OUT = 10

NC = 2
NS = 16
L = 16
NW = NC * NS
EW = E // NW
K = 80
NCH = EW // K
KA = 100
NCHA = EW // KA
CPB = 4
NBLK = NCHA // CPB

ZRT = 624
ZB = 8

TB = 1000
NB = N // TB

_mesh = plsc.VectorSubcoreMesh(core_axis_name="c", subcore_axis_name="s")


@functools.partial(
    pl.kernel,
    out_type=jax.ShapeDtypeStruct((NW, N), jnp.float32),
    mesh=_mesh,
    scratch_types=[
        pltpu.VMEM((NCH, K), jnp.int32),
        pltpu.VMEM((N,), jnp.float32),
    ],
    compiler_params=pltpu.CompilerParams(needs_layout_passes=False),
)
def _sc_degree(dst_hbm, out_hbm, idx_d, degv):
    c = lax.axis_index("c")
    s = lax.axis_index("s")
    wid = s * NC + c
    z16 = jnp.zeros((L,), jnp.float32)
    ones16 = jnp.ones((L,), jnp.float32)

    def zero_row(i, carry):
        degv[pl.ds(i * L, L)] = z16
        return carry

    lax.fori_loop(0, N // L, zero_row, 0)
    pltpu.sync_copy(dst_hbm.at[wid], idx_d)

    def count_chunk(j, carry):
        for t in range(K // L):
            idx16 = idx_d[j, pl.ds(t * L, L)]
            plsc.addupdate_scatter(degv, [idx16], ones16)
        return carry

    lax.fori_loop(0, NCH, count_chunk, 0)
    pltpu.sync_copy(degv, out_hbm.at[wid])


@functools.partial(
    pl.kernel,
    out_type=jax.ShapeDtypeStruct((NC, N, D), jnp.float32),
    mesh=_mesh,
    scratch_types=[
        pltpu.VMEM((3, 2, CPB, KA), jnp.int32),
        pltpu.VMEM((3, KA, D), jnp.float32),
        pltpu.VMEM((ZB, D), jnp.float32),
        pltpu.VMEM_SHARED((N, D), jnp.float32),
        pltpu.SemaphoreType.DMA,
        pltpu.SemaphoreType.DMA,
        pltpu.SemaphoreType.DMA,
    ],
)
def _sc_aggregate(g_hbm, src_hbm, dst_hbm, out_hbm, idx,
                  rows, zbuf, aggS, gsem, ssem, isem):
    c = lax.axis_index("c")
    s = lax.axis_index("s")
    wid = s * NC + c
    z16 = jnp.zeros((L,), jnp.float32)

    def zero_row(i, carry):
        for j in range(D // L):
            zbuf[i, pl.ds(j * L, L)] = z16
        return carry

    lax.fori_loop(0, ZB, zero_row, 0)
    base = s * ZRT

    def zero_acc(t, carry):
        pltpu.sync_copy(zbuf, aggS.at[pl.ds(base + t * ZB, ZB)])
        return carry

    lax.fori_loop(0, ZRT // ZB, zero_acc, 0)

    @pl.when(s == NS - 1)
    def _():
        for t in range((N - NS * ZRT) // ZB):
            pltpu.sync_copy(zbuf, aggS.at[pl.ds(NS * ZRT + t * ZB, ZB)])

    pltpu.sync_copy(src_hbm.at[wid, 0], idx.at[0, 0])
    pltpu.sync_copy(dst_hbm.at[wid, 0], idx.at[0, 1])
    plsc.subcore_barrier()

    def _m3(v):
        return v - 3 * ((v * 43691) >> 17)

    def _gather(j, rb):
        return pltpu.make_async_copy(
            g_hbm.at[idx.at[_m3(j >> 2), 0, j & (CPB - 1)]], rows.at[rb], gsem)

    def _scatter(j, rb):
        return pltpu.make_async_copy(
            rows.at[rb], aggS.at[idx.at[_m3(j >> 2), 1, j & (CPB - 1)]], ssem)

    def _idx_load_s(b):
        return pltpu.make_async_copy(src_hbm.at[wid, b], idx.at[_m3(b), 0],
                                     isem)

    def _idx_load_d(b):
        return pltpu.make_async_copy(dst_hbm.at[wid, b], idx.at[_m3(b), 1],
                                     isem)

    _gather(0, 0).start()
    _gather(1, 1).start()

    def step(j, carry):
        b = j >> 2
        jj = j & (CPB - 1)
        rb = _m3(j)
        rb2 = _m3(j + 2)

        @pl.when((jj == 0) & (b < NBLK - 1))
        def _():
            _idx_load_s(b + 1).start()
            _idx_load_d(b + 1).start()

        _gather(j, rb).wait()
        pltpu.async_copy(rows.at[rb], aggS.at[idx.at[_m3(b), 1, jj]], ssem,
                         add=True)

        @pl.when(j >= 1)
        def _():
            _scatter(j - 1, rb2).wait()

        @pl.when((jj == 2) & (b < NBLK - 1))
        def _():
            _idx_load_s(b + 1).wait()
            _idx_load_d(b + 1).wait()

        @pl.when(j + 2 < NCHA)
        def _():
            _gather(j + 2, rb2).start()

        return carry

    lax.fori_loop(0, NCHA, step, 0)
    _scatter(NCHA - 1, _m3(NCHA - 1)).wait()
    plsc.subcore_barrier()

    pltpu.sync_copy(aggS.at[pl.ds(base, ZRT)],
                    out_hbm.at[c, pl.ds(base, ZRT)])

    @pl.when(s == NS - 1)
    def _():
        pltpu.sync_copy(aggS.at[pl.ds(NS * ZRT, N - NS * ZRT)],
                        out_hbm.at[c, pl.ds(NS * ZRT, N - NS * ZRT)])


def _tc_stage1_body(degT_ref, x_ref, W_ref, g_ref, dinv_ref):
    deg = jnp.sum(degT_ref[...], axis=1) + 1.0
    dinv = lax.rsqrt(deg)
    h = jnp.dot(x_ref[...], W_ref[...], preferred_element_type=jnp.float32)
    g_ref[...] = h * dinv[:, None]
    dinv_ref[...] = dinv[:, None]


_stage1 = pl.pallas_call(
    _tc_stage1_body,
    grid=(NB,),
    in_specs=[
        pl.BlockSpec((TB, NW), lambda i: (i, 0)),
        pl.BlockSpec((TB, D), lambda i: (i, 0)),
        pl.BlockSpec((D, D), lambda i: (0, 0)),
    ],
    out_specs=[
        pl.BlockSpec((TB, D), lambda i: (i, 0)),
        pl.BlockSpec((TB, 1), lambda i: (i, 0)),
    ],
    out_shape=[
        jax.ShapeDtypeStruct((N, D), jnp.float32),
        jax.ShapeDtypeStruct((N, 1), jnp.float32),
    ],
)


def _tc_mid_body(agg_ref, g_ref, dinv_ref, b_ref, W_ref, go_ref):
    dinv = dinv_ref[...]
    a = agg_ref[0] + agg_ref[1] + g_ref[...]
    xin = jnp.maximum(dinv * a + b_ref[...][None, :], 0.0)
    h = jnp.dot(xin, W_ref[...], preferred_element_type=jnp.float32)
    go_ref[...] = h * dinv


_mid = pl.pallas_call(
    _tc_mid_body,
    grid=(NB,),
    in_specs=[
        pl.BlockSpec((NC, TB, D), lambda i: (0, i, 0)),
        pl.BlockSpec((TB, D), lambda i: (i, 0)),
        pl.BlockSpec((TB, 1), lambda i: (i, 0)),
        pl.BlockSpec((D,), lambda i: (0,)),
        pl.BlockSpec((D, D), lambda i: (0, 0)),
    ],
    out_specs=pl.BlockSpec((TB, D), lambda i: (i, 0)),
    out_shape=jax.ShapeDtypeStruct((N, D), jnp.float32),
)


def _tc_final_body(agg_ref, g_ref, dinv_ref, b_ref, batch_ref, Wm0_ref, bm0_ref,
                   Wm1_ref, bm1_ref, emb_ref, logits_ref, probs_ref,
                   acc_ref, cnt_ref):
    i = pl.program_id(0)
    dinv = dinv_ref[...]
    a = agg_ref[0] + agg_ref[1] + g_ref[...]
    emb = jnp.maximum(dinv * a + b_ref[...][None, :], 0.0)
    emb_ref[...] = emb

    @pl.when(i == 0)
    def _():
        acc_ref[...] = jnp.zeros_like(acc_ref)
        cnt_ref[...] = jnp.zeros_like(cnt_ref)

    bat = batch_ref[...][:, 0]
    gids = lax.broadcasted_iota(jnp.int32, (G, TB), 0)
    onehot = (bat[None, :] == gids).astype(jnp.float32)
    acc_ref[...] += jnp.dot(onehot, emb, preferred_element_type=jnp.float32)
    cnt_ref[...] += jnp.sum(onehot, axis=1, keepdims=True)

    @pl.when(i == NB - 1)
    def _():
        pooled = acc_ref[...] / jnp.maximum(cnt_ref[...], 1.0)
        z = jnp.dot(pooled, Wm0_ref[...], preferred_element_type=jnp.float32)
        z = z + bm0_ref[...][None, :]
        z = jnp.where(z > 0, z, jnp.exp(z) - 1.0)
        logits = jnp.dot(z, Wm1_ref[...], preferred_element_type=jnp.float32)
        logits = logits + bm1_ref[...][None, :]
        logits_ref[...] = logits
        m = jnp.max(logits, axis=1, keepdims=True)
        e = jnp.exp(logits - m)
        probs_ref[...] = e / jnp.sum(e, axis=1, keepdims=True)


_final = pl.pallas_call(
    _tc_final_body,
    grid=(NB,),
    in_specs=[
        pl.BlockSpec((NC, TB, D), lambda i: (0, i, 0)),
        pl.BlockSpec((TB, D), lambda i: (i, 0)),
        pl.BlockSpec((TB, 1), lambda i: (i, 0)),
        pl.BlockSpec((D,), lambda i: (0,)),
        pl.BlockSpec((TB, 1), lambda i: (i, 0)),
        pl.BlockSpec((D, H), lambda i: (0, 0)),
        pl.BlockSpec((H,), lambda i: (0,)),
        pl.BlockSpec((H, OUT), lambda i: (0, 0)),
        pl.BlockSpec((OUT,), lambda i: (0,)),
    ],
    out_specs=[
        pl.BlockSpec((TB, D), lambda i: (i, 0)),
        pl.BlockSpec((G, OUT), lambda i: (0, 0)),
        pl.BlockSpec((G, OUT), lambda i: (0, 0)),
    ],
    out_shape=[
        jax.ShapeDtypeStruct((N, D), jnp.float32),
        jax.ShapeDtypeStruct((G, OUT), jnp.float32),
        jax.ShapeDtypeStruct((G, OUT), jnp.float32),
    ],
    scratch_shapes=[
        pltpu.VMEM((G, D), jnp.float32),
        pltpu.VMEM((G, D), jnp.float32),
    ],
)


def kernel(x, edge_index, batch, W1, b1, W2, b2, W3, b3, Wm0, bm0, Wm1, bm1):
    src4 = edge_index[0].reshape(NW, NBLK, CPB, KA)
    dst4 = edge_index[1].reshape(NW, NBLK, CPB, KA)
    dstd = edge_index[1].reshape(NW, NCH, K)
    deg = _sc_degree(dstd)
    g1, dinv = _stage1(deg.T, x, W1)
    agg1 = _sc_aggregate(g1, src4, dst4)
    g2 = _mid(agg1, g1, dinv, b1, W2)
    agg2 = _sc_aggregate(g2, src4, dst4)
    g3 = _mid(agg2, g2, dinv, b2, W3)
    agg3 = _sc_aggregate(g3, src4, dst4)
    emb, logits, probs = _final(agg3, g3, dinv, b3, batch.reshape(N, 1),
                                Wm0, bm0, Wm1, bm1)
    return logits, probs, emb


# speedup vs baseline: 1.0516x; 1.0359x over previous
import functools

import jax
import jax.numpy as jnp
from jax import lax
from jax.experimental import pallas as pl
from jax.experimental.pallas import tpu as pltpu
from jax.experimental.pallas import tpu_sc as plsc

N = 10000
E = 320000
D = 128
G = 64
H = 64
OUT = 10

NC = 2
NS = 16
L = 16
NW = NC * NS
EW = E // NW
K = 80
NCH = EW // K
KA = 100
NCHA = EW // KA
CPB = 4
NBLK = NCHA // CPB

ZRT = 624
ZB = 48

TB = 1000
NB = N // TB

_mesh = plsc.VectorSubcoreMesh(core_axis_name="c", subcore_axis_name="s")


@functools.partial(
    pl.kernel,
    out_type=jax.ShapeDtypeStruct((NW, N), jnp.float32),
    mesh=_mesh,
    scratch_types=[
        pltpu.VMEM((NCH, K), jnp.int32),
        pltpu.VMEM((N,), jnp.float32),
    ],
    compiler_params=pltpu.CompilerParams(needs_layout_passes=False),
)
def _sc_degree(dst_hbm, out_hbm, idx_d, degv):
    c = lax.axis_index("c")
    s = lax.axis_index("s")
    wid = s * NC + c
    z16 = jnp.zeros((L,), jnp.float32)
    ones16 = jnp.ones((L,), jnp.float32)

    def zero_row(i, carry):
        degv[pl.ds(i * L, L)] = z16
        return carry

    lax.fori_loop(0, N // L, zero_row, 0)
    pltpu.sync_copy(dst_hbm.at[wid], idx_d)

    def count_chunk(j, carry):
        for t in range(K // L):
            idx16 = idx_d[j, pl.ds(t * L, L)]
            plsc.addupdate_scatter(degv, [idx16], ones16)
        return carry

    lax.fori_loop(0, NCH, count_chunk, 0)
    pltpu.sync_copy(degv, out_hbm.at[wid])


@functools.partial(
    pl.kernel,
    out_type=jax.ShapeDtypeStruct((NC, N, D), jnp.float32),
    mesh=_mesh,
    scratch_types=[
        pltpu.VMEM((3, 2, CPB, KA), jnp.int32),
        pltpu.VMEM((3, KA, D), jnp.float32),
        pltpu.VMEM((ZB, D), jnp.float32),
        pltpu.VMEM_SHARED((N, D), jnp.float32),
        pltpu.SemaphoreType.DMA,
        pltpu.SemaphoreType.DMA,
        pltpu.SemaphoreType.DMA,
        pltpu.SemaphoreType.DMA,
    ],
)
def _sc_aggregate(g_hbm, src_hbm, dst_hbm, out_hbm, idx,
                  rows, zbuf, aggS, gsem, ssem, isem, zsem):
    c = lax.axis_index("c")
    s = lax.axis_index("s")
    wid = s * NC + c
    z16 = jnp.zeros((L,), jnp.float32)

    def zero_row(i, carry):
        for j in range(D // L):
            zbuf[i, pl.ds(j * L, L)] = z16
        return carry

    lax.fori_loop(0, ZB, zero_row, 0)
    base = s * ZRT

    def _m3(v):
        return v - 3 * ((v * 43691) >> 17)

    def _gather(j, rb):
        return pltpu.make_async_copy(
            g_hbm.at[idx.at[_m3(j >> 2), 0, j & (CPB - 1)]], rows.at[rb], gsem)

    def _scatter(j, rb):
        return pltpu.make_async_copy(
            rows.at[rb], aggS.at[idx.at[_m3(j >> 2), 1, j & (CPB - 1)]], ssem)

    def _idx_load_s(b):
        return pltpu.make_async_copy(src_hbm.at[wid, b], idx.at[_m3(b), 0],
                                     isem)

    def _idx_load_d(b):
        return pltpu.make_async_copy(dst_hbm.at[wid, b], idx.at[_m3(b), 1],
                                     isem)

    def _zero(t):
        return pltpu.make_async_copy(zbuf, aggS.at[pl.ds(base + t * ZB, ZB)],
                                     zsem)

    def _zero_tail():
        return pltpu.make_async_copy(
            zbuf.at[pl.ds(0, N - NS * ZRT)],
            aggS.at[pl.ds(NS * ZRT, N - NS * ZRT)], zsem)

    pltpu.sync_copy(src_hbm.at[wid, 0], idx.at[0, 0])
    pltpu.sync_copy(dst_hbm.at[wid, 0], idx.at[0, 1])
    _gather(0, 0).start()
    _gather(1, 1).start()

    def zero_fire(t, carry):
        _zero(t).start()
        return carry

    lax.fori_loop(0, ZRT // ZB, zero_fire, 0)

    @pl.when(s == NS - 1)
    def _():
        _zero_tail().start()

    def zero_drain(t, carry):
        _zero(t).wait()
        return carry

    lax.fori_loop(0, ZRT // ZB, zero_drain, 0)

    @pl.when(s == NS - 1)
    def _():
        _zero_tail().wait()

    plsc.subcore_barrier()


    def step(j, carry):
        b = j >> 2
        jj = j & (CPB - 1)
        rb = _m3(j)
        rb2 = _m3(j + 2)

        @pl.when((jj == 0) & (b < NBLK - 1))
        def _():
            _idx_load_s(b + 1).start()
            _idx_load_d(b + 1).start()

        _gather(j, rb).wait()
        pltpu.async_copy(rows.at[rb], aggS.at[idx.at[_m3(b), 1, jj]], ssem,
                         add=True)

        @pl.when(j >= 1)
        def _():
            _scatter(j - 1, rb2).wait()

        @pl.when((jj == 2) & (b < NBLK - 1))
        def _():
            _idx_load_s(b + 1).wait()
            _idx_load_d(b + 1).wait()

        @pl.when(j + 2 < NCHA)
        def _():
            _gather(j + 2, rb2).start()

        return carry

    lax.fori_loop(0, NCHA, step, 0)
    _scatter(NCHA - 1, _m3(NCHA - 1)).wait()
    plsc.subcore_barrier()

    pltpu.sync_copy(aggS.at[pl.ds(base, ZRT)],
                    out_hbm.at[c, pl.ds(base, ZRT)])

    @pl.when(s == NS - 1)
    def _():
        pltpu.sync_copy(aggS.at[pl.ds(NS * ZRT, N - NS * ZRT)],
                        out_hbm.at[c, pl.ds(NS * ZRT, N - NS * ZRT)])


def _tc_stage1_body(degT_ref, x_ref, W_ref, g_ref, dinv_ref):
    deg = jnp.sum(degT_ref[...], axis=1) + 1.0
    dinv = lax.rsqrt(deg)
    h = jnp.dot(x_ref[...], W_ref[...], preferred_element_type=jnp.float32)
    g_ref[...] = h * dinv[:, None]
    dinv_ref[...] = dinv[:, None]


_stage1 = pl.pallas_call(
    _tc_stage1_body,
    grid=(NB,),
    in_specs=[
        pl.BlockSpec((TB, NW), lambda i: (i, 0)),
        pl.BlockSpec((TB, D), lambda i: (i, 0)),
        pl.BlockSpec((D, D), lambda i: (0, 0)),
    ],
    out_specs=[
        pl.BlockSpec((TB, D), lambda i: (i, 0)),
        pl.BlockSpec((TB, 1), lambda i: (i, 0)),
    ],
    out_shape=[
        jax.ShapeDtypeStruct((N, D), jnp.float32),
        jax.ShapeDtypeStruct((N, 1), jnp.float32),
    ],
)


def _tc_mid_body(agg_ref, g_ref, dinv_ref, b_ref, W_ref, go_ref):
    dinv = dinv_ref[...]
    a = agg_ref[0] + agg_ref[1] + g_ref[...]
    xin = jnp.maximum(dinv * a + b_ref[...][None, :], 0.0)
    h = jnp.dot(xin, W_ref[...], preferred_element_type=jnp.float32)
    go_ref[...] = h * dinv


_mid = pl.pallas_call(
    _tc_mid_body,
    grid=(NB,),
    in_specs=[
        pl.BlockSpec((NC, TB, D), lambda i: (0, i, 0)),
        pl.BlockSpec((TB, D), lambda i: (i, 0)),
        pl.BlockSpec((TB, 1), lambda i: (i, 0)),
        pl.BlockSpec((D,), lambda i: (0,)),
        pl.BlockSpec((D, D), lambda i: (0, 0)),
    ],
    out_specs=pl.BlockSpec((TB, D), lambda i: (i, 0)),
    out_shape=jax.ShapeDtypeStruct((N, D), jnp.float32),
)


def _tc_final_body(agg_ref, g_ref, dinv_ref, b_ref, batch_ref, Wm0_ref, bm0_ref,
                   Wm1_ref, bm1_ref, emb_ref, logits_ref, probs_ref,
                   acc_ref, cnt_ref):
    i = pl.program_id(0)
    dinv = dinv_ref[...]
    a = agg_ref[0] + agg_ref[1] + g_ref[...]
    emb = jnp.maximum(dinv * a + b_ref[...][None, :], 0.0)
    emb_ref[...] = emb

    @pl.when(i == 0)
    def _():
        acc_ref[...] = jnp.zeros_like(acc_ref)
        cnt_ref[...] = jnp.zeros_like(cnt_ref)

    bat = batch_ref[...][:, 0]
    gids = lax.broadcasted_iota(jnp.int32, (G, TB), 0)
    onehot = (bat[None, :] == gids).astype(jnp.float32)
    acc_ref[...] += jnp.dot(onehot, emb, preferred_element_type=jnp.float32)
    cnt_ref[...] += jnp.sum(onehot, axis=1, keepdims=True)

    @pl.when(i == NB - 1)
    def _():
        pooled = acc_ref[...] / jnp.maximum(cnt_ref[...], 1.0)
        z = jnp.dot(pooled, Wm0_ref[...], preferred_element_type=jnp.float32)
        z = z + bm0_ref[...][None, :]
        z = jnp.where(z > 0, z, jnp.exp(z) - 1.0)
        logits = jnp.dot(z, Wm1_ref[...], preferred_element_type=jnp.float32)
        logits = logits + bm1_ref[...][None, :]
        logits_ref[...] = logits
        m = jnp.max(logits, axis=1, keepdims=True)
        e = jnp.exp(logits - m)
        probs_ref[...] = e / jnp.sum(e, axis=1, keepdims=True)


_final = pl.pallas_call(
    _tc_final_body,
    grid=(NB,),
    in_specs=[
        pl.BlockSpec((NC, TB, D), lambda i: (0, i, 0)),
        pl.BlockSpec((TB, D), lambda i: (i, 0)),
        pl.BlockSpec((TB, 1), lambda i: (i, 0)),
        pl.BlockSpec((D,), lambda i: (0,)),
        pl.BlockSpec((TB, 1), lambda i: (i, 0)),
        pl.BlockSpec((D, H), lambda i: (0, 0)),
        pl.BlockSpec((H,), lambda i: (0,)),
        pl.BlockSpec((H, OUT), lambda i: (0, 0)),
        pl.BlockSpec((OUT,), lambda i: (0,)),
    ],
    out_specs=[
        pl.BlockSpec((TB, D), lambda i: (i, 0)),
        pl.BlockSpec((G, OUT), lambda i: (0, 0)),
        pl.BlockSpec((G, OUT), lambda i: (0, 0)),
    ],
    out_shape=[
        jax.ShapeDtypeStruct((N, D), jnp.float32),
        jax.ShapeDtypeStruct((G, OUT), jnp.float32),
        jax.ShapeDtypeStruct((G, OUT), jnp.float32),
    ],
    scratch_shapes=[
        pltpu.VMEM((G, D), jnp.float32),
        pltpu.VMEM((G, D), jnp.float32),
    ],
)


def kernel(x, edge_index, batch, W1, b1, W2, b2, W3, b3, Wm0, bm0, Wm1, bm1):
    src4 = edge_index[0].reshape(NW, NBLK, CPB, KA)
    dst4 = edge_index[1].reshape(NW, NBLK, CPB, KA)
    dstd = edge_index[1].reshape(NW, NCH, K)
    deg = _sc_degree(dstd)
    g1, dinv = _stage1(deg.T, x, W1)
    agg1 = _sc_aggregate(g1, src4, dst4)
    g2 = _mid(agg1, g1, dinv, b1, W2)
    agg2 = _sc_aggregate(g2, src4, dst4)
    g3 = _mid(agg2, g2, dinv, b2, W3)
    agg3 = _sc_aggregate(g3, src4, dst4)
    emb, logits, probs = _final(agg3, g3, dinv, b3, batch.reshape(N, 1),
                                Wm0, bm0, Wm1, bm1)
    return logits, probs, emb
